# Initial kernel scaffold; baseline (speedup 1.0000x reference)
import jax
import jax.numpy as jnp
from jax.experimental import pallas as pl


def kernel(a_C, a_F, b_C, b_F, W1, b1, W2, b2):
    raise NotImplementedError("write your pallas kernel here")


# speedup vs baseline: 11.6124x; 11.6124x over previous
import functools

import jax
import jax.numpy as jnp
from jax import lax
from jax.experimental import pallas as pl
from jax.experimental.pallas import tpu as pltpu
from jax.experimental.pallas import tpu_sc as plsc

NPB = 4096
TOPK = 16
FULL_SCALE = 128
D2_CLAMP = 4096
AUGD = 128
TILE_TOPK = 256
TILE_ROWS = 512
NWORKERS = 32
SC_CHUNK = 64

_HI = lax.Precision.HIGHEST


def _prep_body(af_ref, bf_ref, wa_ref, wu_ref, b1_ref, a_ref, u_ref):
    a_ref[...] = (
        jnp.dot(af_ref[...], wa_ref[...], precision=_HI,
                preferred_element_type=jnp.float32)
        + b1_ref[...]
    )
    u_ref[...] = jnp.dot(bf_ref[...], wu_ref[...], precision=_HI,
                         preferred_element_type=jnp.float32)


def _topk_body(aaug_ref, baug_ref, idx_ref, w_ref, *, tiles_per_batch):
    b = pl.program_id(0) // tiles_per_batch
    d2 = jnp.dot(aaug_ref[...], baug_ref[0], precision=_HI,
                 preferred_element_type=jnp.float32)
    d2 = jnp.minimum(d2, float(D2_CLAMP))
    keys = d2.astype(jnp.int32) * NPB + lax.broadcasted_iota(
        jnp.int32, d2.shape, 1)
    rows = d2.shape[0]
    sel = jnp.zeros((rows, TOPK), jnp.int32)
    lane = lax.broadcasted_iota(jnp.int32, (rows, TOPK), 1)
    for it in range(TOPK):
        m = jnp.min(keys, axis=1, keepdims=True)
        sel = jnp.where(lane == it, m, sel)
        keys = jnp.where(keys == m, jnp.iinfo(jnp.int32).max, keys)
    col = sel & (NPB - 1)
    d2i = sel >> 12
    w_ref[...] = 0.5 - jnp.sqrt(d2i.astype(jnp.float32)) * (1.0 / FULL_SCALE)
    idx_ref[...] = col + b * NPB


def _final_body(af_ref, s_ref, w2t_ref, b2_ref, out_ref):
    d = af_ref.shape[1]
    tail = (
        jnp.dot(s_ref[...], w2t_ref[...], precision=_HI,
                preferred_element_type=jnp.float32)
        + float(TOPK) * b2_ref[...]
    )
    out_ref[:, :d] = af_ref[...]
    out_ref[:, d:] = tail


def _sc_combine_body(a_hbm, u_hbm, idx_hbm, w_hbm, s_hbm,
                     a_v, s_v, idx_v, w_v, g_v, sem, *, rows_per_worker, d):
    wid = lax.axis_index("s") * 2 + lax.axis_index("c")
    nvec = d // 16

    def chunk_body(c, _):
        base = wid * rows_per_worker + c * SC_CHUNK
        pltpu.sync_copy(a_hbm.at[pl.ds(base, SC_CHUNK)], a_v)
        pltpu.sync_copy(idx_hbm.at[pl.ds(base, SC_CHUNK)], idx_v)
        pltpu.sync_copy(w_hbm.at[pl.ds(base * TOPK, SC_CHUNK * TOPK)], w_v)

        def point_body(p, _):
            idx_vec = idx_v[p]
            pltpu.async_copy(u_hbm.at[idx_vec], g_v, sem).wait()
            a_regs = [a_v[p, pl.ds(v * 16, 16)] for v in range(nvec)]
            accs = [jnp.zeros((16,), jnp.float32) for _ in range(nvec)]
            w_row = w_v[pl.ds(p * TOPK, TOPK)]
            for j in range(TOPK):
                wj = jnp.full((16,), w_row[j], jnp.float32)
                for v in range(nvec):
                    h = jnp.maximum(a_regs[v] + g_v[j, pl.ds(v * 16, 16)], 0.0)
                    accs[v] = accs[v] + wj * h
            for v in range(nvec):
                s_v[p, pl.ds(v * 16, 16)] = accs[v]
            return _

        lax.fori_loop(0, SC_CHUNK, point_body, None)
        pltpu.sync_copy(s_v, s_hbm.at[pl.ds(base, SC_CHUNK)])
        return _

    lax.fori_loop(0, rows_per_worker // SC_CHUNK, chunk_body, None)


@functools.lru_cache(maxsize=None)
def _build_calls(n, d):
    nb = n // NPB
    f32 = jnp.float32

    prep = pl.pallas_call(
        _prep_body,
        grid=(n // TILE_ROWS,),
        in_specs=[
            pl.BlockSpec((TILE_ROWS, d), lambda i: (i, 0)),
            pl.BlockSpec((TILE_ROWS, d), lambda i: (i, 0)),
            pl.BlockSpec((d, d), lambda i: (0, 0)),
            pl.BlockSpec((d, d), lambda i: (0, 0)),
            pl.BlockSpec((1, d), lambda i: (0, 0)),
        ],
        out_specs=[
            pl.BlockSpec((TILE_ROWS, d), lambda i: (i, 0)),
            pl.BlockSpec((TILE_ROWS, d), lambda i: (i, 0)),
        ],
        out_shape=[
            jax.ShapeDtypeStruct((n, d), f32),
            jax.ShapeDtypeStruct((n, d), f32),
        ],
    )

    tiles_per_batch = NPB // TILE_TOPK
    topk = pl.pallas_call(
        functools.partial(_topk_body, tiles_per_batch=tiles_per_batch),
        grid=(n // TILE_TOPK,),
        in_specs=[
            pl.BlockSpec((TILE_TOPK, AUGD), lambda i: (i, 0)),
            pl.BlockSpec((1, AUGD, NPB),
                         lambda i, tpb=tiles_per_batch: (i // tpb, 0, 0)),
        ],
        out_specs=[
            pl.BlockSpec((TILE_TOPK, TOPK), lambda i: (i, 0)),
            pl.BlockSpec((TILE_TOPK, TOPK), lambda i: (i, 0)),
        ],
        out_shape=[
            jax.ShapeDtypeStruct((n, TOPK), jnp.int32),
            jax.ShapeDtypeStruct((n, TOPK), f32),
        ],
    )

    final = pl.pallas_call(
        _final_body,
        grid=(n // TILE_ROWS,),
        in_specs=[
            pl.BlockSpec((TILE_ROWS, d), lambda i: (i, 0)),
            pl.BlockSpec((TILE_ROWS, d), lambda i: (i, 0)),
            pl.BlockSpec((d, d), lambda i: (0, 0)),
            pl.BlockSpec((1, d), lambda i: (0, 0)),
        ],
        out_specs=pl.BlockSpec((TILE_ROWS, 2 * d), lambda i: (i, 0)),
        out_shape=jax.ShapeDtypeStruct((n, 2 * d), f32),
    )

    rows_per_worker = n // NWORKERS
    mesh = plsc.VectorSubcoreMesh(
        core_axis_name="c", subcore_axis_name="s",
        num_cores=2, num_subcores=16)
    sc_combine = pl.kernel(
        functools.partial(_sc_combine_body,
                          rows_per_worker=rows_per_worker, d=d),
        out_type=jax.ShapeDtypeStruct((n, d), f32),
        mesh=mesh,
        scratch_types=[
            pltpu.VMEM((SC_CHUNK, d), f32),
            pltpu.VMEM((SC_CHUNK, d), f32),
            pltpu.VMEM((SC_CHUNK, TOPK), jnp.int32),
            pltpu.VMEM((SC_CHUNK * TOPK,), f32),
            pltpu.VMEM((TOPK, d), f32),
            pltpu.SemaphoreType.DMA,
        ],
    )

    return prep, topk, sc_combine, final, nb


def kernel(a_C, a_F, b_C, b_F, W1, b1, W2, b2):
    n, d = a_F.shape
    prep, topk, sc_combine, final, nb = _build_calls(n, d)
    f32 = jnp.float32

    w1t = W1.T
    wa = w1t[d:]
    wu = w1t[:d] - wa
    A, U = prep(a_F, b_F, wa, wu, b1.reshape(1, d))

    ca = (a_C[:, 1:4] // 16).astype(f32)
    cb = (b_C[:, 1:4] // 16).astype(f32)
    x2 = jnp.sum(ca * ca, axis=1)
    y2 = jnp.sum(cb * cb, axis=1)
    a_aug = jnp.zeros((n, AUGD), f32)
    a_aug = a_aug.at[:, 0:3].set(ca).at[:, 3].set(x2).at[:, 4].set(1.0)
    b_aug = jnp.zeros((n, AUGD), f32)
    b_aug = b_aug.at[:, 0:3].set(-2.0 * cb).at[:, 3].set(1.0).at[:, 4].set(y2)
    b_aug_t = b_aug.reshape(nb, NPB, AUGD).transpose(0, 2, 1)

    idxg, w = topk(a_aug, b_aug_t)
    S = sc_combine(A, U, idxg, w.reshape(-1))
    return final(a_F, S, W2.T, b2.reshape(1, d))


# speedup vs baseline: 13.3559x; 1.1501x over previous
import functools

import jax
import jax.numpy as jnp
from jax import lax
from jax.experimental import pallas as pl
from jax.experimental.pallas import tpu as pltpu
from jax.experimental.pallas import tpu_sc as plsc

NPB = 4096
TOPK = 16
FULL_SCALE = 128
D2_CLAMP = 4096
AUGD = 128
TILE_TOPK = 256
TILE_ROWS = 512
NWORKERS = 32
SC_CHUNK = 128
SC_GP = 2

_HI = lax.Precision.HIGHEST


def _prep_body(af_ref, bf_ref, wa_ref, wu_ref, b1_ref, a_ref, u_ref):
    a_ref[...] = (
        jnp.dot(af_ref[...], wa_ref[...], precision=_HI,
                preferred_element_type=jnp.float32)
        + b1_ref[...]
    )
    u_ref[...] = jnp.dot(bf_ref[...], wu_ref[...], precision=_HI,
                         preferred_element_type=jnp.float32)


def _topk_body(aaug_ref, baug_ref, idx_ref, w_ref, *, tiles_per_batch):
    b = pl.program_id(0) // tiles_per_batch
    d2 = jnp.dot(aaug_ref[...], baug_ref[0], precision=_HI,
                 preferred_element_type=jnp.float32)
    d2 = jnp.minimum(d2, float(D2_CLAMP))
    keys = d2.astype(jnp.int32) * NPB + lax.broadcasted_iota(
        jnp.int32, d2.shape, 1)
    rows = d2.shape[0]
    sel = jnp.zeros((rows, TOPK), jnp.int32)
    lane = lax.broadcasted_iota(jnp.int32, (rows, TOPK), 1)
    for it in range(TOPK):
        m = jnp.min(keys, axis=1, keepdims=True)
        sel = jnp.where(lane == it, m, sel)
        keys = jnp.where(keys == m, jnp.iinfo(jnp.int32).max, keys)
    col = sel & (NPB - 1)
    d2i = sel >> 12
    w_ref[...] = 0.5 - jnp.sqrt(d2i.astype(jnp.float32)) * (1.0 / FULL_SCALE)
    idx_ref[...] = col + b * NPB


def _final_body(af_ref, s_ref, w2t_ref, b2_ref, out_ref):
    d = af_ref.shape[1]
    tail = (
        jnp.dot(s_ref[...], w2t_ref[...], precision=_HI,
                preferred_element_type=jnp.float32)
        + float(TOPK) * b2_ref[...]
    )
    out_ref[:, :d] = af_ref[...]
    out_ref[:, d:] = tail


def _sc_combine_body(a_hbm, u_hbm, idx_hbm, w_hbm, s_hbm,
                     a_v, s_v, idx_v, w_v, g_v, sem0, sem1,
                     *, rows_per_worker, d):
    wid = lax.axis_index("s") * 2 + lax.axis_index("c")
    nvec = d // 16
    ngroups = SC_CHUNK // SC_GP

    def chunk_body(c, _):
        base = wid * rows_per_worker + c * SC_CHUNK
        pltpu.sync_copy(a_hbm.at[pl.ds(base, SC_CHUNK)], a_v)
        pltpu.sync_copy(idx_hbm.at[pl.ds(base * TOPK, SC_CHUNK * TOPK)],
                        idx_v)
        pltpu.sync_copy(w_hbm.at[pl.ds(base * TOPK, SC_CHUNK * TOPK)], w_v)

        def issue(gr, slot, sem):
            return pltpu.async_copy(
                u_hbm.at[idx_v.at[pl.ds(gr * SC_GP * TOPK, SC_GP * TOPK)]],
                g_v.at[slot], sem)

        def wait_group(slot, sem):
            pltpu.make_async_copy(
                u_hbm.at[idx_v.at[pl.ds(0, SC_GP * TOPK)]],
                g_v.at[slot], sem).wait()

        def compute_group(gr, slot):
            for pp in range(SC_GP):
                p = gr * SC_GP + pp
                a_regs = [a_v[p, pl.ds(v * 16, 16)] for v in range(nvec)]
                w_row = w_v[pl.ds(p * TOPK, TOPK)]
                wj = jnp.full((16,), w_row[0], jnp.float32)
                accs = [wj * jnp.maximum(
                    a_regs[v] + g_v[slot, pp * TOPK, pl.ds(v * 16, 16)], 0.0)
                    for v in range(nvec)]
                for j in range(1, TOPK):
                    wj = jnp.full((16,), w_row[j], jnp.float32)
                    for v in range(nvec):
                        h = jnp.maximum(
                            a_regs[v]
                            + g_v[slot, pp * TOPK + j, pl.ds(v * 16, 16)],
                            0.0)
                        accs[v] = accs[v] + wj * h
                for v in range(nvec):
                    s_v[p, pl.ds(v * 16, 16)] = accs[v]

        issue(0, 0, sem0)
        issue(1, 1, sem1)

        def pair_body(g, _):
            wait_group(0, sem0)
            compute_group(2 * g, 0)

            @pl.when(2 * g + 2 < ngroups)
            def _refill0():
                issue(2 * g + 2, 0, sem0)

            wait_group(1, sem1)
            compute_group(2 * g + 1, 1)

            @pl.when(2 * g + 3 < ngroups)
            def _refill1():
                issue(2 * g + 3, 1, sem1)

            return _

        lax.fori_loop(0, ngroups // 2, pair_body, None)
        pltpu.sync_copy(s_v, s_hbm.at[pl.ds(base, SC_CHUNK)])
        return _

    lax.fori_loop(0, rows_per_worker // SC_CHUNK, chunk_body, None)


@functools.lru_cache(maxsize=None)
def _build_calls(n, d):
    nb = n // NPB
    f32 = jnp.float32

    prep = pl.pallas_call(
        _prep_body,
        grid=(n // TILE_ROWS,),
        in_specs=[
            pl.BlockSpec((TILE_ROWS, d), lambda i: (i, 0)),
            pl.BlockSpec((TILE_ROWS, d), lambda i: (i, 0)),
            pl.BlockSpec((d, d), lambda i: (0, 0)),
            pl.BlockSpec((d, d), lambda i: (0, 0)),
            pl.BlockSpec((1, d), lambda i: (0, 0)),
        ],
        out_specs=[
            pl.BlockSpec((TILE_ROWS, d), lambda i: (i, 0)),
            pl.BlockSpec((TILE_ROWS, d), lambda i: (i, 0)),
        ],
        out_shape=[
            jax.ShapeDtypeStruct((n, d), f32),
            jax.ShapeDtypeStruct((n, d), f32),
        ],
    )

    tiles_per_batch = NPB // TILE_TOPK
    topk = pl.pallas_call(
        functools.partial(_topk_body, tiles_per_batch=tiles_per_batch),
        grid=(n // TILE_TOPK,),
        in_specs=[
            pl.BlockSpec((TILE_TOPK, AUGD), lambda i: (i, 0)),
            pl.BlockSpec((1, AUGD, NPB),
                         lambda i, tpb=tiles_per_batch: (i // tpb, 0, 0)),
        ],
        out_specs=[
            pl.BlockSpec((TILE_TOPK, TOPK), lambda i: (i, 0)),
            pl.BlockSpec((TILE_TOPK, TOPK), lambda i: (i, 0)),
        ],
        out_shape=[
            jax.ShapeDtypeStruct((n, TOPK), jnp.int32),
            jax.ShapeDtypeStruct((n, TOPK), f32),
        ],
    )

    final = pl.pallas_call(
        _final_body,
        grid=(n // TILE_ROWS,),
        in_specs=[
            pl.BlockSpec((TILE_ROWS, d), lambda i: (i, 0)),
            pl.BlockSpec((TILE_ROWS, d), lambda i: (i, 0)),
            pl.BlockSpec((d, d), lambda i: (0, 0)),
            pl.BlockSpec((1, d), lambda i: (0, 0)),
        ],
        out_specs=pl.BlockSpec((TILE_ROWS, 2 * d), lambda i: (i, 0)),
        out_shape=jax.ShapeDtypeStruct((n, 2 * d), f32),
    )

    rows_per_worker = n // NWORKERS
    mesh = plsc.VectorSubcoreMesh(
        core_axis_name="c", subcore_axis_name="s",
        num_cores=2, num_subcores=16)
    sc_combine = pl.kernel(
        functools.partial(_sc_combine_body,
                          rows_per_worker=rows_per_worker, d=d),
        out_type=jax.ShapeDtypeStruct((n, d), f32),
        mesh=mesh,
        scratch_types=[
            pltpu.VMEM((SC_CHUNK, d), f32),
            pltpu.VMEM((SC_CHUNK, d), f32),
            pltpu.VMEM((SC_CHUNK * TOPK,), jnp.int32),
            pltpu.VMEM((SC_CHUNK * TOPK,), f32),
            pltpu.VMEM((2, SC_GP * TOPK, d), f32),
            pltpu.SemaphoreType.DMA,
            pltpu.SemaphoreType.DMA,
        ],
    )

    return prep, topk, sc_combine, final, nb


def kernel(a_C, a_F, b_C, b_F, W1, b1, W2, b2):
    n, d = a_F.shape
    prep, topk, sc_combine, final, nb = _build_calls(n, d)
    f32 = jnp.float32

    w1t = W1.T
    wa = w1t[d:]
    wu = w1t[:d] - wa
    A, U = prep(a_F, b_F, wa, wu, b1.reshape(1, d))

    ca = (a_C[:, 1:4] // 16).astype(f32)
    cb = (b_C[:, 1:4] // 16).astype(f32)
    x2 = jnp.sum(ca * ca, axis=1)
    y2 = jnp.sum(cb * cb, axis=1)
    a_aug = jnp.zeros((n, AUGD), f32)
    a_aug = a_aug.at[:, 0:3].set(ca).at[:, 3].set(x2).at[:, 4].set(1.0)
    b_aug = jnp.zeros((n, AUGD), f32)
    b_aug = b_aug.at[:, 0:3].set(-2.0 * cb).at[:, 3].set(1.0).at[:, 4].set(y2)
    b_aug_t = b_aug.reshape(nb, NPB, AUGD).transpose(0, 2, 1)

    idxg, w = topk(a_aug, b_aug_t)
    S = sc_combine(A, U, idxg.reshape(-1), w.reshape(-1))
    return final(a_F, S, W2.T, b2.reshape(1, d))


# speedup vs baseline: 14.9276x; 1.1177x over previous
import functools

import jax
import jax.numpy as jnp
from jax import lax
from jax.experimental import pallas as pl
from jax.experimental.pallas import tpu as pltpu
from jax.experimental.pallas import tpu_sc as plsc

NPB = 4096
TOPK = 16
FULL_SCALE = 128
D2_CLAMP = 4096
AUGD = 128
TILE_TOPK = 256
TILE_ROWS = 512
NWORKERS = 32
SC_CHUNK = 128
SC_GP = 2

_HI = lax.Precision.HIGHEST


def _prep_body(af_ref, bf_ref, wa_ref, wu_ref, b1_ref, a_ref, u_ref):
    a_ref[...] = (
        jnp.dot(af_ref[...], wa_ref[...], precision=_HI,
                preferred_element_type=jnp.float32)
        + b1_ref[...]
    )
    u_ref[...] = jnp.dot(bf_ref[...], wu_ref[...], precision=_HI,
                         preferred_element_type=jnp.float32)


def _topk_body(aaug_ref, baug_ref, idx_ref, w_ref, *, tiles_per_batch):
    b = pl.program_id(0) // tiles_per_batch
    d2 = jnp.dot(aaug_ref[...], baug_ref[0], precision=_HI,
                 preferred_element_type=jnp.float32)
    d2 = jnp.minimum(d2, float(D2_CLAMP))
    keys = d2.astype(jnp.int32) * NPB + lax.broadcasted_iota(
        jnp.int32, d2.shape, 1)
    rows = d2.shape[0]
    q = NPB // 4
    s0, s1, s2, s3 = (keys[:, i * q:(i + 1) * q] for i in range(4))

    def _cx(a, b):
        return jnp.minimum(a, b), jnp.maximum(a, b)

    s0, s1 = _cx(s0, s1)
    s2, s3 = _cx(s2, s3)
    s0, s2 = _cx(s0, s2)
    s1, s3 = _cx(s1, s3)
    s1, s2 = _cx(s1, s2)
    sel = jnp.zeros((rows, TOPK), jnp.int32)
    lane = lax.broadcasted_iota(jnp.int32, (rows, TOPK), 1)
    for it in range(TOPK):
        m = jnp.min(s0, axis=1, keepdims=True)
        sel = jnp.where(lane == it, m, sel)
        if it < TOPK - 1:
            hit = s0 == m
            s0 = jnp.where(hit, s1, s0)
            s1 = jnp.where(hit, s2, s1)
            s2 = jnp.where(hit, s3, s2)
            s3 = jnp.where(hit, jnp.iinfo(jnp.int32).max, s3)
    col = sel & (NPB - 1)
    d2i = sel >> 12
    w_ref[...] = 0.5 - jnp.sqrt(d2i.astype(jnp.float32)) * (1.0 / FULL_SCALE)
    idx_ref[...] = col + b * NPB


def _final_body(af_ref, s_ref, w2t_ref, b2_ref, out_ref):
    d = af_ref.shape[1]
    tail = (
        jnp.dot(s_ref[...], w2t_ref[...], precision=_HI,
                preferred_element_type=jnp.float32)
        + float(TOPK) * b2_ref[...]
    )
    out_ref[:, :d] = af_ref[...]
    out_ref[:, d:] = tail


def _sc_combine_body(a_hbm, u_hbm, idx_hbm, w_hbm, s_hbm,
                     a_v, s_v, idx_v, w_v, g_v, sem0, sem1,
                     *, rows_per_worker, d):
    wid = lax.axis_index("s") * 2 + lax.axis_index("c")
    nvec = d // 16
    ngroups = SC_CHUNK // SC_GP

    def chunk_body(c, _):
        base = wid * rows_per_worker + c * SC_CHUNK
        pltpu.sync_copy(a_hbm.at[pl.ds(base, SC_CHUNK)], a_v)
        pltpu.sync_copy(idx_hbm.at[pl.ds(base * TOPK, SC_CHUNK * TOPK)],
                        idx_v)
        pltpu.sync_copy(w_hbm.at[pl.ds(base * TOPK, SC_CHUNK * TOPK)], w_v)

        def issue(gr, slot, sem):
            return pltpu.async_copy(
                u_hbm.at[idx_v.at[pl.ds(gr * SC_GP * TOPK, SC_GP * TOPK)]],
                g_v.at[slot], sem)

        def wait_group(slot, sem):
            pltpu.make_async_copy(
                u_hbm.at[idx_v.at[pl.ds(0, SC_GP * TOPK)]],
                g_v.at[slot], sem).wait()

        def compute_group(gr, slot):
            for pp in range(SC_GP):
                p = gr * SC_GP + pp
                a_regs = [a_v[p, pl.ds(v * 16, 16)] for v in range(nvec)]
                w_row = w_v[pl.ds(p * TOPK, TOPK)]
                wj = jnp.full((16,), w_row[0], jnp.float32)
                accs = [wj * jnp.maximum(
                    a_regs[v] + g_v[slot, pp * TOPK, pl.ds(v * 16, 16)], 0.0)
                    for v in range(nvec)]
                for j in range(1, TOPK):
                    wj = jnp.full((16,), w_row[j], jnp.float32)
                    for v in range(nvec):
                        h = jnp.maximum(
                            a_regs[v]
                            + g_v[slot, pp * TOPK + j, pl.ds(v * 16, 16)],
                            0.0)
                        accs[v] = accs[v] + wj * h
                for v in range(nvec):
                    s_v[p, pl.ds(v * 16, 16)] = accs[v]

        issue(0, 0, sem0)
        issue(1, 1, sem1)

        def pair_body(g, _):
            wait_group(0, sem0)
            compute_group(2 * g, 0)

            @pl.when(2 * g + 2 < ngroups)
            def _refill0():
                issue(2 * g + 2, 0, sem0)

            wait_group(1, sem1)
            compute_group(2 * g + 1, 1)

            @pl.when(2 * g + 3 < ngroups)
            def _refill1():
                issue(2 * g + 3, 1, sem1)

            return _

        lax.fori_loop(0, ngroups // 2, pair_body, None)
        pltpu.sync_copy(s_v, s_hbm.at[pl.ds(base, SC_CHUNK)])
        return _

    lax.fori_loop(0, rows_per_worker // SC_CHUNK, chunk_body, None)


@functools.lru_cache(maxsize=None)
def _build_calls(n, d):
    nb = n // NPB
    f32 = jnp.float32

    prep = pl.pallas_call(
        _prep_body,
        grid=(n // TILE_ROWS,),
        in_specs=[
            pl.BlockSpec((TILE_ROWS, d), lambda i: (i, 0)),
            pl.BlockSpec((TILE_ROWS, d), lambda i: (i, 0)),
            pl.BlockSpec((d, d), lambda i: (0, 0)),
            pl.BlockSpec((d, d), lambda i: (0, 0)),
            pl.BlockSpec((1, d), lambda i: (0, 0)),
        ],
        out_specs=[
            pl.BlockSpec((TILE_ROWS, d), lambda i: (i, 0)),
            pl.BlockSpec((TILE_ROWS, d), lambda i: (i, 0)),
        ],
        out_shape=[
            jax.ShapeDtypeStruct((n, d), f32),
            jax.ShapeDtypeStruct((n, d), f32),
        ],
    )

    tiles_per_batch = NPB // TILE_TOPK
    topk = pl.pallas_call(
        functools.partial(_topk_body, tiles_per_batch=tiles_per_batch),
        grid=(n // TILE_TOPK,),
        in_specs=[
            pl.BlockSpec((TILE_TOPK, AUGD), lambda i: (i, 0)),
            pl.BlockSpec((1, AUGD, NPB),
                         lambda i, tpb=tiles_per_batch: (i // tpb, 0, 0)),
        ],
        out_specs=[
            pl.BlockSpec((TILE_TOPK, TOPK), lambda i: (i, 0)),
            pl.BlockSpec((TILE_TOPK, TOPK), lambda i: (i, 0)),
        ],
        out_shape=[
            jax.ShapeDtypeStruct((n, TOPK), jnp.int32),
            jax.ShapeDtypeStruct((n, TOPK), f32),
        ],
    )

    final = pl.pallas_call(
        _final_body,
        grid=(n // TILE_ROWS,),
        in_specs=[
            pl.BlockSpec((TILE_ROWS, d), lambda i: (i, 0)),
            pl.BlockSpec((TILE_ROWS, d), lambda i: (i, 0)),
            pl.BlockSpec((d, d), lambda i: (0, 0)),
            pl.BlockSpec((1, d), lambda i: (0, 0)),
        ],
        out_specs=pl.BlockSpec((TILE_ROWS, 2 * d), lambda i: (i, 0)),
        out_shape=jax.ShapeDtypeStruct((n, 2 * d), f32),
    )

    rows_per_worker = n // NWORKERS
    mesh = plsc.VectorSubcoreMesh(
        core_axis_name="c", subcore_axis_name="s",
        num_cores=2, num_subcores=16)
    sc_combine = pl.kernel(
        functools.partial(_sc_combine_body,
                          rows_per_worker=rows_per_worker, d=d),
        out_type=jax.ShapeDtypeStruct((n, d), f32),
        mesh=mesh,
        scratch_types=[
            pltpu.VMEM((SC_CHUNK, d), f32),
            pltpu.VMEM((SC_CHUNK, d), f32),
            pltpu.VMEM((SC_CHUNK * TOPK,), jnp.int32),
            pltpu.VMEM((SC_CHUNK * TOPK,), f32),
            pltpu.VMEM((2, SC_GP * TOPK, d), f32),
            pltpu.SemaphoreType.DMA,
            pltpu.SemaphoreType.DMA,
        ],
    )

    return prep, topk, sc_combine, final, nb


def kernel(a_C, a_F, b_C, b_F, W1, b1, W2, b2):
    n, d = a_F.shape
    prep, topk, sc_combine, final, nb = _build_calls(n, d)
    f32 = jnp.float32

    w1t = W1.T
    wa = w1t[d:]
    wu = w1t[:d] - wa
    A, U = prep(a_F, b_F, wa, wu, b1.reshape(1, d))

    ca = (a_C[:, 1:4] // 16).astype(f32)
    cb = (b_C[:, 1:4] // 16).astype(f32)
    x2 = jnp.sum(ca * ca, axis=1)
    y2 = jnp.sum(cb * cb, axis=1)
    a_aug = jnp.zeros((n, AUGD), f32)
    a_aug = a_aug.at[:, 0:3].set(ca).at[:, 3].set(x2).at[:, 4].set(1.0)
    b_aug = jnp.zeros((n, AUGD), f32)
    b_aug = b_aug.at[:, 0:3].set(-2.0 * cb).at[:, 3].set(1.0).at[:, 4].set(y2)
    b_aug_t = b_aug.reshape(nb, NPB, AUGD).transpose(0, 2, 1)

    idxg, w = topk(a_aug, b_aug_t)
    S = sc_combine(A, U, idxg.reshape(-1), w.reshape(-1))
    return final(a_F, S, W2.T, b2.reshape(1, d))


# speedup vs baseline: 18.6123x; 1.2468x over previous
import functools

import jax
import jax.numpy as jnp
from jax import lax
from jax.experimental import pallas as pl
from jax.experimental.pallas import tpu as pltpu
from jax.experimental.pallas import tpu_sc as plsc

NPB = 4096
TOPK = 16
FULL_SCALE = 128
D2_CLAMP = 4096
CKD = 16
TILE_TOPK = 256
TILE_ROWS = 512
NWORKERS = 32
SC_CHUNK = 128
SC_GP = 2

_HI = lax.Precision.HIGHEST


def _prep_body(af_ref, bf_ref, wa_ref, wu_ref, b1_ref, a_ref, u_ref):
    a_ref[...] = (
        jnp.dot(af_ref[...], wa_ref[...], precision=_HI,
                preferred_element_type=jnp.float32)
        + b1_ref[...]
    )
    u_ref[...] = jnp.dot(bf_ref[...], wu_ref[...], precision=_HI,
                         preferred_element_type=jnp.float32)


def _topk_body(acb_ref, x2_ref, bct_ref, y2_ref, idx_ref, w_ref,
               *, tiles_per_batch):
    b = pl.program_id(0) // tiles_per_batch
    prod = jnp.dot(acb_ref[...], bct_ref[0],
                   preferred_element_type=jnp.float32)
    d2 = x2_ref[...] + (y2_ref[0] + prod)
    d2 = jnp.minimum(d2, float(D2_CLAMP))
    keys = d2.astype(jnp.int32) * NPB + lax.broadcasted_iota(
        jnp.int32, d2.shape, 1)
    rows = d2.shape[0]
    q = NPB // 4
    s0, s1, s2, s3 = (keys[:, i * q:(i + 1) * q] for i in range(4))

    def _cx(a, b):
        return jnp.minimum(a, b), jnp.maximum(a, b)

    s0, s1 = _cx(s0, s1)
    s2, s3 = _cx(s2, s3)
    s0, s2 = _cx(s0, s2)
    s1, s3 = _cx(s1, s3)
    s1, s2 = _cx(s1, s2)
    sel = jnp.zeros((rows, TOPK), jnp.int32)
    lane = lax.broadcasted_iota(jnp.int32, (rows, TOPK), 1)
    for it in range(TOPK):
        m = jnp.min(s0, axis=1, keepdims=True)
        sel = jnp.where(lane == it, m, sel)
        if it < TOPK - 1:
            hit = s0 == m
            s0 = jnp.where(hit, s1, s0)
            s1 = jnp.where(hit, s2, s1)
            s2 = jnp.where(hit, s3, s2)
            s3 = jnp.where(hit, jnp.iinfo(jnp.int32).max, s3)
    col = sel & (NPB - 1)
    d2i = sel >> 12
    w_ref[...] = 0.5 - jnp.sqrt(d2i.astype(jnp.float32)) * (1.0 / FULL_SCALE)
    idx_ref[...] = col + b * NPB


def _final_body(af_ref, s_ref, w2t_ref, b2_ref, out_ref):
    d = af_ref.shape[1]
    tail = (
        jnp.dot(s_ref[...], w2t_ref[...], precision=_HI,
                preferred_element_type=jnp.float32)
        + float(TOPK) * b2_ref[...]
    )
    out_ref[:, :d] = af_ref[...]
    out_ref[:, d:] = tail


def _sc_combine_body(a_hbm, u_hbm, idx_hbm, w_hbm, s_hbm,
                     a_v, s_v, idx_v, w_v, g_v, sem0, sem1,
                     *, rows_per_worker, d):
    wid = lax.axis_index("s") * 2 + lax.axis_index("c")
    nvec = d // 16
    ngroups = SC_CHUNK // SC_GP

    def chunk_body(c, _):
        base = wid * rows_per_worker + c * SC_CHUNK
        pltpu.sync_copy(a_hbm.at[pl.ds(base, SC_CHUNK)], a_v)
        pltpu.sync_copy(idx_hbm.at[pl.ds(base * TOPK, SC_CHUNK * TOPK)],
                        idx_v)
        pltpu.sync_copy(w_hbm.at[pl.ds(base * TOPK, SC_CHUNK * TOPK)], w_v)

        def issue(gr, slot, sem):
            return pltpu.async_copy(
                u_hbm.at[idx_v.at[pl.ds(gr * SC_GP * TOPK, SC_GP * TOPK)]],
                g_v.at[slot], sem)

        def wait_group(slot, sem):
            pltpu.make_async_copy(
                u_hbm.at[idx_v.at[pl.ds(0, SC_GP * TOPK)]],
                g_v.at[slot], sem).wait()

        def compute_group(gr, slot):
            for pp in range(SC_GP):
                p = gr * SC_GP + pp
                a_regs = [a_v[p, pl.ds(v * 16, 16)] for v in range(nvec)]
                w_row = w_v[pl.ds(p * TOPK, TOPK)]
                wj = jnp.full((16,), w_row[0], jnp.float32)
                accs = [wj * jnp.maximum(
                    a_regs[v] + g_v[slot, pp * TOPK, pl.ds(v * 16, 16)], 0.0)
                    for v in range(nvec)]
                for j in range(1, TOPK):
                    wj = jnp.full((16,), w_row[j], jnp.float32)
                    for v in range(nvec):
                        h = jnp.maximum(
                            a_regs[v]
                            + g_v[slot, pp * TOPK + j, pl.ds(v * 16, 16)],
                            0.0)
                        accs[v] = accs[v] + wj * h
                for v in range(nvec):
                    s_v[p, pl.ds(v * 16, 16)] = accs[v]

        issue(0, 0, sem0)
        issue(1, 1, sem1)

        def pair_body(g, _):
            wait_group(0, sem0)
            compute_group(2 * g, 0)

            @pl.when(2 * g + 2 < ngroups)
            def _refill0():
                issue(2 * g + 2, 0, sem0)

            wait_group(1, sem1)
            compute_group(2 * g + 1, 1)

            @pl.when(2 * g + 3 < ngroups)
            def _refill1():
                issue(2 * g + 3, 1, sem1)

            return _

        lax.fori_loop(0, ngroups // 2, pair_body, None)
        pltpu.sync_copy(s_v, s_hbm.at[pl.ds(base, SC_CHUNK)])
        return _

    lax.fori_loop(0, rows_per_worker // SC_CHUNK, chunk_body, None)


@functools.lru_cache(maxsize=None)
def _build_calls(n, d):
    nb = n // NPB
    f32 = jnp.float32

    prep = pl.pallas_call(
        _prep_body,
        grid=(n // TILE_ROWS,),
        in_specs=[
            pl.BlockSpec((TILE_ROWS, d), lambda i: (i, 0)),
            pl.BlockSpec((TILE_ROWS, d), lambda i: (i, 0)),
            pl.BlockSpec((d, d), lambda i: (0, 0)),
            pl.BlockSpec((d, d), lambda i: (0, 0)),
            pl.BlockSpec((1, d), lambda i: (0, 0)),
        ],
        out_specs=[
            pl.BlockSpec((TILE_ROWS, d), lambda i: (i, 0)),
            pl.BlockSpec((TILE_ROWS, d), lambda i: (i, 0)),
        ],
        out_shape=[
            jax.ShapeDtypeStruct((n, d), f32),
            jax.ShapeDtypeStruct((n, d), f32),
        ],
    )

    tiles_per_batch = NPB // TILE_TOPK
    topk = pl.pallas_call(
        functools.partial(_topk_body, tiles_per_batch=tiles_per_batch),
        grid=(n // TILE_TOPK,),
        in_specs=[
            pl.BlockSpec((TILE_TOPK, CKD), lambda i: (i, 0)),
            pl.BlockSpec((TILE_TOPK, 1), lambda i: (i, 0)),
            pl.BlockSpec((1, CKD, NPB),
                         lambda i, tpb=tiles_per_batch: (i // tpb, 0, 0)),
            pl.BlockSpec((1, 1, NPB),
                         lambda i, tpb=tiles_per_batch: (i // tpb, 0, 0)),
        ],
        out_specs=[
            pl.BlockSpec((TILE_TOPK, TOPK), lambda i: (i, 0)),
            pl.BlockSpec((TILE_TOPK, TOPK), lambda i: (i, 0)),
        ],
        out_shape=[
            jax.ShapeDtypeStruct((n, TOPK), jnp.int32),
            jax.ShapeDtypeStruct((n, TOPK), f32),
        ],
    )

    final = pl.pallas_call(
        _final_body,
        grid=(n // TILE_ROWS,),
        in_specs=[
            pl.BlockSpec((TILE_ROWS, d), lambda i: (i, 0)),
            pl.BlockSpec((TILE_ROWS, d), lambda i: (i, 0)),
            pl.BlockSpec((d, d), lambda i: (0, 0)),
            pl.BlockSpec((1, d), lambda i: (0, 0)),
        ],
        out_specs=pl.BlockSpec((TILE_ROWS, 2 * d), lambda i: (i, 0)),
        out_shape=jax.ShapeDtypeStruct((n, 2 * d), f32),
    )

    rows_per_worker = n // NWORKERS
    mesh = plsc.VectorSubcoreMesh(
        core_axis_name="c", subcore_axis_name="s",
        num_cores=2, num_subcores=16)
    sc_combine = pl.kernel(
        functools.partial(_sc_combine_body,
                          rows_per_worker=rows_per_worker, d=d),
        out_type=jax.ShapeDtypeStruct((n, d), f32),
        mesh=mesh,
        scratch_types=[
            pltpu.VMEM((SC_CHUNK, d), f32),
            pltpu.VMEM((SC_CHUNK, d), f32),
            pltpu.VMEM((SC_CHUNK * TOPK,), jnp.int32),
            pltpu.VMEM((SC_CHUNK * TOPK,), f32),
            pltpu.VMEM((2, SC_GP * TOPK, d), f32),
            pltpu.SemaphoreType.DMA,
            pltpu.SemaphoreType.DMA,
        ],
    )

    return prep, topk, sc_combine, final, nb


def kernel(a_C, a_F, b_C, b_F, W1, b1, W2, b2):
    n, d = a_F.shape
    prep, topk, sc_combine, final, nb = _build_calls(n, d)
    f32 = jnp.float32

    w1t = W1.T
    wa = w1t[d:]
    wu = w1t[:d] - wa
    A, U = prep(a_F, b_F, wa, wu, b1.reshape(1, d))

    ca = (a_C[:, 1:4] // 16).astype(f32)
    cb = (b_C[:, 1:4] // 16).astype(f32)
    x2 = jnp.sum(ca * ca, axis=1).reshape(n, 1)
    y2 = jnp.sum(cb * cb, axis=1).reshape(nb, 1, NPB)
    a_cb = jnp.zeros((n, CKD), jnp.bfloat16)
    a_cb = a_cb.at[:, 0:3].set(ca.astype(jnp.bfloat16))
    b_ct = jnp.zeros((n, CKD), jnp.bfloat16)
    b_ct = b_ct.at[:, 0:3].set((-2.0 * cb).astype(jnp.bfloat16))
    b_ct = b_ct.reshape(nb, NPB, CKD).transpose(0, 2, 1)

    idxg, w = topk(a_cb, x2, b_ct, y2)
    S = sc_combine(A, U, idxg.reshape(-1), w.reshape(-1))
    return final(a_F, S, W2.T, b2.reshape(1, d))


# speedup vs baseline: 22.6431x; 1.2166x over previous
import functools

import jax
import jax.numpy as jnp
from jax import lax
from jax.experimental import pallas as pl
from jax.experimental.pallas import tpu as pltpu
from jax.experimental.pallas import tpu_sc as plsc

NPB = 4096
TOPK = 16
FULL_SCALE = 128
D2_CLAMP = 4096
CKD = 16
TILE_TOPK = 256
TILE_ROWS = 512
NWORKERS = 32
SC_CHUNK = 128
SC_GP = 2

_HI = lax.Precision.HIGHEST


def _prep_body(af_ref, bf_ref, wa_ref, wu_ref, b1_ref, a_ref, u_ref):
    a_ref[...] = (
        jnp.dot(af_ref[...], wa_ref[...], precision=_HI,
                preferred_element_type=jnp.float32)
        + b1_ref[...]
    )
    u_ref[...] = jnp.dot(bf_ref[...], wu_ref[...], precision=_HI,
                         preferred_element_type=jnp.float32)


def _topk_body(acb_ref, x2_ref, bct_ref, y2_ref, idx_ref, w_ref):
    prod = jnp.dot(acb_ref[...], bct_ref[...],
                   preferred_element_type=jnp.float32)
    d2 = x2_ref[...] + (y2_ref[...] + prod)
    d2 = jnp.minimum(d2, float(D2_CLAMP))
    keys = d2.astype(jnp.int32) * NPB + lax.broadcasted_iota(
        jnp.int32, d2.shape, 1)
    rows = d2.shape[0]
    q = NPB // 4
    s0, s1, s2, s3 = (keys[:, i * q:(i + 1) * q] for i in range(4))

    def _cx(a, b):
        return jnp.minimum(a, b), jnp.maximum(a, b)

    s0, s1 = _cx(s0, s1)
    s2, s3 = _cx(s2, s3)
    s0, s2 = _cx(s0, s2)
    s1, s3 = _cx(s1, s3)
    s1, s2 = _cx(s1, s2)
    sel = jnp.zeros((rows, TOPK), jnp.int32)
    lane = lax.broadcasted_iota(jnp.int32, (rows, TOPK), 1)
    for it in range(TOPK):
        m = jnp.min(s0, axis=1, keepdims=True)
        sel = jnp.where(lane == it, m, sel)
        if it < TOPK - 1:
            hit = s0 == m
            s0 = jnp.where(hit, s1, s0)
            s1 = jnp.where(hit, s2, s1)
            s2 = jnp.where(hit, s3, s2)
            s3 = jnp.where(hit, jnp.iinfo(jnp.int32).max, s3)
    col = sel & (NPB - 1)
    d2i = sel >> 12
    w_ref[...] = 0.5 - jnp.sqrt(d2i.astype(jnp.float32)) * (1.0 / FULL_SCALE)
    idx_ref[...] = col


def _final_body(af_ref, s_ref, w2t_ref, b2_ref, out_ref):
    d = af_ref.shape[1]
    tail = (
        jnp.dot(s_ref[...], w2t_ref[...], precision=_HI,
                preferred_element_type=jnp.float32)
        + float(TOPK) * b2_ref[...]
    )
    out_ref[:, :d] = af_ref[...]
    out_ref[:, d:] = tail


def _sc_combine_body(a_hbm, u_hbm, idx_hbm, w_hbm, s_hbm,
                     a_v, s_v, idx_v, w_v, g_v, sem0, sem1,
                     *, rows_per_worker, d):
    wid = lax.axis_index("s") * 2 + lax.axis_index("c")
    nvec = d // 16
    ngroups = SC_CHUNK // SC_GP

    def chunk_body(c, _):
        base = wid * rows_per_worker + c * SC_CHUNK
        pltpu.sync_copy(a_hbm.at[pl.ds(base, SC_CHUNK)], a_v)
        pltpu.sync_copy(idx_hbm.at[pl.ds(base * TOPK, SC_CHUNK * TOPK)],
                        idx_v)
        pltpu.sync_copy(w_hbm.at[pl.ds(base * TOPK, SC_CHUNK * TOPK)], w_v)

        def issue(gr, slot, sem):
            return pltpu.async_copy(
                u_hbm.at[idx_v.at[pl.ds(gr * SC_GP * TOPK, SC_GP * TOPK)]],
                g_v.at[slot], sem)

        def wait_group(slot, sem):
            pltpu.make_async_copy(
                u_hbm.at[idx_v.at[pl.ds(0, SC_GP * TOPK)]],
                g_v.at[slot], sem).wait()

        def compute_group(gr, slot):
            for pp in range(SC_GP):
                p = gr * SC_GP + pp
                a_regs = [a_v[p, pl.ds(v * 16, 16)] for v in range(nvec)]
                w_row = w_v[pl.ds(p * TOPK, TOPK)]
                wj = jnp.full((16,), w_row[0], jnp.float32)
                accs = [wj * jnp.maximum(
                    a_regs[v] + g_v[slot, pp * TOPK, pl.ds(v * 16, 16)], 0.0)
                    for v in range(nvec)]
                for j in range(1, TOPK):
                    wj = jnp.full((16,), w_row[j], jnp.float32)
                    for v in range(nvec):
                        h = jnp.maximum(
                            a_regs[v]
                            + g_v[slot, pp * TOPK + j, pl.ds(v * 16, 16)],
                            0.0)
                        accs[v] = accs[v] + wj * h
                for v in range(nvec):
                    s_v[p, pl.ds(v * 16, 16)] = accs[v]

        issue(0, 0, sem0)
        issue(1, 1, sem1)

        def pair_body(g, _):
            wait_group(0, sem0)
            compute_group(2 * g, 0)

            @pl.when(2 * g + 2 < ngroups)
            def _refill0():
                issue(2 * g + 2, 0, sem0)

            wait_group(1, sem1)
            compute_group(2 * g + 1, 1)

            @pl.when(2 * g + 3 < ngroups)
            def _refill1():
                issue(2 * g + 3, 1, sem1)

            return _

        lax.fori_loop(0, ngroups // 2, pair_body, None)
        pltpu.sync_copy(s_v, s_hbm.at[pl.ds(base, SC_CHUNK)])
        return _

    lax.fori_loop(0, rows_per_worker // SC_CHUNK, chunk_body, None)


@functools.lru_cache(maxsize=None)
def _build_calls(n, d):
    nb = n // NPB
    f32 = jnp.float32

    prep = pl.pallas_call(
        _prep_body,
        grid=(n // TILE_ROWS,),
        in_specs=[
            pl.BlockSpec((TILE_ROWS, d), lambda i: (i, 0)),
            pl.BlockSpec((TILE_ROWS, d), lambda i: (i, 0)),
            pl.BlockSpec((d, d), lambda i: (0, 0)),
            pl.BlockSpec((d, d), lambda i: (0, 0)),
            pl.BlockSpec((1, d), lambda i: (0, 0)),
        ],
        out_specs=[
            pl.BlockSpec((TILE_ROWS, d), lambda i: (i, 0)),
            pl.BlockSpec((TILE_ROWS, d), lambda i: (i, 0)),
        ],
        out_shape=[
            jax.ShapeDtypeStruct((n, d), f32),
            jax.ShapeDtypeStruct((n, d), f32),
        ],
    )

    topk = pl.pallas_call(
        _topk_body,
        grid=(NPB // TILE_TOPK,),
        in_specs=[
            pl.BlockSpec((TILE_TOPK, CKD), lambda i: (i, 0)),
            pl.BlockSpec((TILE_TOPK, 1), lambda i: (i, 0)),
            pl.BlockSpec((CKD, NPB), lambda i: (0, 0)),
            pl.BlockSpec((1, NPB), lambda i: (0, 0)),
        ],
        out_specs=[
            pl.BlockSpec((TILE_TOPK, TOPK), lambda i: (i, 0)),
            pl.BlockSpec((TILE_TOPK, TOPK), lambda i: (i, 0)),
        ],
        out_shape=[
            jax.ShapeDtypeStruct((NPB, TOPK), jnp.int32),
            jax.ShapeDtypeStruct((NPB, TOPK), f32),
        ],
    )

    final = pl.pallas_call(
        _final_body,
        grid=(NPB // TILE_ROWS,),
        in_specs=[
            pl.BlockSpec((TILE_ROWS, d), lambda i: (i, 0)),
            pl.BlockSpec((TILE_ROWS, d), lambda i: (i, 0)),
            pl.BlockSpec((d, d), lambda i: (0, 0)),
            pl.BlockSpec((1, d), lambda i: (0, 0)),
        ],
        out_specs=pl.BlockSpec((TILE_ROWS, 2 * d), lambda i: (i, 0)),
        out_shape=jax.ShapeDtypeStruct((NPB, 2 * d), f32),
    )

    rows_per_worker = NPB // NWORKERS
    mesh = plsc.VectorSubcoreMesh(
        core_axis_name="c", subcore_axis_name="s",
        num_cores=2, num_subcores=16)
    sc_combine = pl.kernel(
        functools.partial(_sc_combine_body,
                          rows_per_worker=rows_per_worker, d=d),
        out_type=jax.ShapeDtypeStruct((NPB, d), f32),
        mesh=mesh,
        scratch_types=[
            pltpu.VMEM((SC_CHUNK, d), f32),
            pltpu.VMEM((SC_CHUNK, d), f32),
            pltpu.VMEM((SC_CHUNK * TOPK,), jnp.int32),
            pltpu.VMEM((SC_CHUNK * TOPK,), f32),
            pltpu.VMEM((2, SC_GP * TOPK, d), f32),
            pltpu.SemaphoreType.DMA,
            pltpu.SemaphoreType.DMA,
        ],
    )

    return prep, topk, sc_combine, final, nb


def kernel(a_C, a_F, b_C, b_F, W1, b1, W2, b2):
    n, d = a_F.shape
    prep, topk, sc_combine, final, nb = _build_calls(n, d)
    f32 = jnp.float32

    w1t = W1.T
    wa = w1t[d:]
    wu = w1t[:d] - wa
    A, U = prep(a_F, b_F, wa, wu, b1.reshape(1, d))

    ca = (a_C[:, 1:4] // 16).astype(f32)
    cb = (b_C[:, 1:4] // 16).astype(f32)
    x2 = jnp.sum(ca * ca, axis=1).reshape(n, 1)
    y2 = jnp.sum(cb * cb, axis=1).reshape(nb, 1, NPB)
    a_cb = jnp.zeros((n, CKD), jnp.bfloat16)
    a_cb = a_cb.at[:, 0:3].set(ca.astype(jnp.bfloat16))
    b_ct = jnp.zeros((n, CKD), jnp.bfloat16)
    b_ct = b_ct.at[:, 0:3].set((-2.0 * cb).astype(jnp.bfloat16))
    b_ct = b_ct.reshape(nb, NPB, CKD).transpose(0, 2, 1)

    w2t = W2.T
    b2r = b2.reshape(1, d)
    outs = []
    for b in range(nb):
        sl = slice(b * NPB, (b + 1) * NPB)
        idx_b, w_b = topk(a_cb[sl], x2[sl], b_ct[b], y2[b])
        S_b = sc_combine(A[sl], U[sl], idx_b.reshape(-1), w_b.reshape(-1))
        outs.append(final(a_F[sl], S_b, w2t, b2r))
    return jnp.concatenate(outs, axis=0)


# speedup vs baseline: 23.2513x; 1.0269x over previous
import functools

import jax
import jax.numpy as jnp
from jax import lax
from jax.experimental import pallas as pl
from jax.experimental.pallas import tpu as pltpu
from jax.experimental.pallas import tpu_sc as plsc

NPB = 4096
TOPK = 16
FULL_SCALE = 128
D2_CLAMP = 4096
CKD = 16
TILE_TOPK = 256
TILE_ROWS = 512
NWORKERS = 32
UNIT = 2048
SC_GP = 2

_HI = lax.Precision.HIGHEST


def _prep_body(af_ref, bf_ref, wa_ref, wu_ref, b1_ref, a_ref, u_ref):
    a_ref[...] = (
        jnp.dot(af_ref[...], wa_ref[...], precision=_HI,
                preferred_element_type=jnp.float32)
        + b1_ref[...]
    )
    u_ref[...] = jnp.dot(bf_ref[...], wu_ref[...], precision=_HI,
                         preferred_element_type=jnp.float32)


def _topk_body(acb_ref, x2_ref, bct_ref, y2_ref, idx_ref, w_ref):
    prod = jnp.dot(acb_ref[...], bct_ref[...],
                   preferred_element_type=jnp.float32)
    d2 = x2_ref[...] + (y2_ref[...] + prod)
    d2 = jnp.minimum(d2, float(D2_CLAMP))
    keys = d2.astype(jnp.int32) * NPB + lax.broadcasted_iota(
        jnp.int32, d2.shape, 1)
    rows = d2.shape[0]
    q = NPB // 8
    s = [keys[:, i * q:(i + 1) * q] for i in range(8)]

    def _cx(i, j):
        s[i], s[j] = jnp.minimum(s[i], s[j]), jnp.maximum(s[i], s[j])

    for (i, j) in [(0, 1), (2, 3), (4, 5), (6, 7),
                   (0, 2), (1, 3), (4, 6), (5, 7),
                   (1, 2), (5, 6),
                   (0, 4), (1, 5), (2, 6), (3, 7),
                   (2, 4), (3, 5),
                   (1, 2), (3, 4), (5, 6)]:
        _cx(i, j)
    sel = jnp.zeros((rows, TOPK), jnp.int32)
    lane = lax.broadcasted_iota(jnp.int32, (rows, TOPK), 1)
    for it in range(TOPK):
        m = jnp.min(s[0], axis=1, keepdims=True)
        sel = jnp.where(lane == it, m, sel)
        if it < TOPK - 1:
            hit = s[0] == m
            for lvl in range(7):
                s[lvl] = jnp.where(hit, s[lvl + 1], s[lvl])
            s[7] = jnp.where(hit, jnp.iinfo(jnp.int32).max, s[7])
    col = sel & (NPB - 1)
    d2i = sel >> 12
    w_ref[...] = 0.5 - jnp.sqrt(d2i.astype(jnp.float32)) * (1.0 / FULL_SCALE)
    idx_ref[...] = col


def _final_body(af_ref, s_ref, w2t_ref, b2_ref, out_ref):
    d = af_ref.shape[1]
    tail = (
        jnp.dot(s_ref[...], w2t_ref[...], precision=_HI,
                preferred_element_type=jnp.float32)
        + float(TOPK) * b2_ref[...]
    )
    out_ref[:, :d] = af_ref[...]
    out_ref[:, d:] = tail


def _sc_combine_body(a_hbm, u_hbm, idx_hbm, w_hbm, s_hbm,
                     a_v, s_v, idx_v, w_v, g_v, sem0, sem1,
                     *, rows_per_worker, d, chunk):
    wid = lax.axis_index("s") * 2 + lax.axis_index("c")
    nvec = d // 16
    ngroups = chunk // SC_GP

    def chunk_body(c, _):
        base = wid * rows_per_worker + c * chunk
        pltpu.sync_copy(a_hbm.at[pl.ds(base, chunk)], a_v)
        pltpu.sync_copy(idx_hbm.at[pl.ds(base * TOPK, chunk * TOPK)],
                        idx_v)
        pltpu.sync_copy(w_hbm.at[pl.ds(base * TOPK, chunk * TOPK)], w_v)

        def issue(gr, slot, sem):
            return pltpu.async_copy(
                u_hbm.at[idx_v.at[pl.ds(gr * SC_GP * TOPK, SC_GP * TOPK)]],
                g_v.at[slot], sem)

        def wait_group(slot, sem):
            pltpu.make_async_copy(
                u_hbm.at[idx_v.at[pl.ds(0, SC_GP * TOPK)]],
                g_v.at[slot], sem).wait()

        def compute_group(gr, slot):
            for pp in range(SC_GP):
                p = gr * SC_GP + pp
                a_regs = [a_v[p, pl.ds(v * 16, 16)] for v in range(nvec)]
                w_row = w_v[pl.ds(p * TOPK, TOPK)]
                wj = jnp.full((16,), w_row[0], jnp.float32)
                accs = [wj * jnp.maximum(
                    a_regs[v] + g_v[slot, pp * TOPK, pl.ds(v * 16, 16)], 0.0)
                    for v in range(nvec)]
                for j in range(1, TOPK):
                    wj = jnp.full((16,), w_row[j], jnp.float32)
                    for v in range(nvec):
                        h = jnp.maximum(
                            a_regs[v]
                            + g_v[slot, pp * TOPK + j, pl.ds(v * 16, 16)],
                            0.0)
                        accs[v] = accs[v] + wj * h
                for v in range(nvec):
                    s_v[p, pl.ds(v * 16, 16)] = accs[v]

        issue(0, 0, sem0)
        issue(1, 1, sem1)

        def pair_body(g, _):
            wait_group(0, sem0)
            compute_group(2 * g, 0)

            @pl.when(2 * g + 2 < ngroups)
            def _refill0():
                issue(2 * g + 2, 0, sem0)

            wait_group(1, sem1)
            compute_group(2 * g + 1, 1)

            @pl.when(2 * g + 3 < ngroups)
            def _refill1():
                issue(2 * g + 3, 1, sem1)

            return _

        lax.fori_loop(0, ngroups // 2, pair_body, None)
        pltpu.sync_copy(s_v, s_hbm.at[pl.ds(base, chunk)])
        return _

    lax.fori_loop(0, rows_per_worker // chunk, chunk_body, None)


@functools.lru_cache(maxsize=None)
def _build_calls(n, d):
    nb = n // NPB
    f32 = jnp.float32

    prep = pl.pallas_call(
        _prep_body,
        grid=(n // TILE_ROWS,),
        in_specs=[
            pl.BlockSpec((TILE_ROWS, d), lambda i: (i, 0)),
            pl.BlockSpec((TILE_ROWS, d), lambda i: (i, 0)),
            pl.BlockSpec((d, d), lambda i: (0, 0)),
            pl.BlockSpec((d, d), lambda i: (0, 0)),
            pl.BlockSpec((1, d), lambda i: (0, 0)),
        ],
        out_specs=[
            pl.BlockSpec((TILE_ROWS, d), lambda i: (i, 0)),
            pl.BlockSpec((TILE_ROWS, d), lambda i: (i, 0)),
        ],
        out_shape=[
            jax.ShapeDtypeStruct((n, d), f32),
            jax.ShapeDtypeStruct((n, d), f32),
        ],
    )

    topk = pl.pallas_call(
        _topk_body,
        grid=(UNIT // TILE_TOPK,),
        in_specs=[
            pl.BlockSpec((TILE_TOPK, CKD), lambda i: (i, 0)),
            pl.BlockSpec((TILE_TOPK, 1), lambda i: (i, 0)),
            pl.BlockSpec((CKD, NPB), lambda i: (0, 0)),
            pl.BlockSpec((1, NPB), lambda i: (0, 0)),
        ],
        out_specs=[
            pl.BlockSpec((TILE_TOPK, TOPK), lambda i: (i, 0)),
            pl.BlockSpec((TILE_TOPK, TOPK), lambda i: (i, 0)),
        ],
        out_shape=[
            jax.ShapeDtypeStruct((UNIT, TOPK), jnp.int32),
            jax.ShapeDtypeStruct((UNIT, TOPK), f32),
        ],
    )

    final = pl.pallas_call(
        _final_body,
        grid=(UNIT // TILE_ROWS,),
        in_specs=[
            pl.BlockSpec((TILE_ROWS, d), lambda i: (i, 0)),
            pl.BlockSpec((TILE_ROWS, d), lambda i: (i, 0)),
            pl.BlockSpec((d, d), lambda i: (0, 0)),
            pl.BlockSpec((1, d), lambda i: (0, 0)),
        ],
        out_specs=pl.BlockSpec((TILE_ROWS, 2 * d), lambda i: (i, 0)),
        out_shape=jax.ShapeDtypeStruct((UNIT, 2 * d), f32),
    )

    rows_per_worker = UNIT // NWORKERS
    chunk = rows_per_worker
    mesh = plsc.VectorSubcoreMesh(
        core_axis_name="c", subcore_axis_name="s",
        num_cores=2, num_subcores=16)
    sc_combine = pl.kernel(
        functools.partial(_sc_combine_body,
                          rows_per_worker=rows_per_worker, d=d,
                          chunk=chunk),
        out_type=jax.ShapeDtypeStruct((UNIT, d), f32),
        mesh=mesh,
        scratch_types=[
            pltpu.VMEM((chunk, d), f32),
            pltpu.VMEM((chunk, d), f32),
            pltpu.VMEM((chunk * TOPK,), jnp.int32),
            pltpu.VMEM((chunk * TOPK,), f32),
            pltpu.VMEM((2, SC_GP * TOPK, d), f32),
            pltpu.SemaphoreType.DMA,
            pltpu.SemaphoreType.DMA,
        ],
    )

    return prep, topk, sc_combine, final, nb


def kernel(a_C, a_F, b_C, b_F, W1, b1, W2, b2):
    n, d = a_F.shape
    prep, topk, sc_combine, final, nb = _build_calls(n, d)
    f32 = jnp.float32

    w1t = W1.T
    wa = w1t[d:]
    wu = w1t[:d] - wa
    A, U = prep(a_F, b_F, wa, wu, b1.reshape(1, d))

    ca = (a_C[:, 1:4] // 16).astype(f32)
    cb = (b_C[:, 1:4] // 16).astype(f32)
    x2 = jnp.sum(ca * ca, axis=1).reshape(n, 1)
    y2 = jnp.sum(cb * cb, axis=1).reshape(nb, 1, NPB)
    a_cb = jnp.zeros((n, CKD), jnp.bfloat16)
    a_cb = a_cb.at[:, 0:3].set(ca.astype(jnp.bfloat16))
    b_ct = jnp.zeros((n, CKD), jnp.bfloat16)
    b_ct = b_ct.at[:, 0:3].set((-2.0 * cb).astype(jnp.bfloat16))
    b_ct = b_ct.reshape(nb, NPB, CKD).transpose(0, 2, 1)

    w2t = W2.T
    b2r = b2.reshape(1, d)
    outs = []
    for u in range(n // UNIT):
        sl = slice(u * UNIT, (u + 1) * UNIT)
        b = (u * UNIT) // NPB
        bsl = slice(b * NPB, (b + 1) * NPB)
        idx_u, w_u = topk(a_cb[sl], x2[sl], b_ct[b], y2[b])
        S_u = sc_combine(A[sl], U[bsl], idx_u.reshape(-1), w_u.reshape(-1))
        outs.append(final(a_F[sl], S_u, w2t, b2r))
    return jnp.concatenate(outs, axis=0)
